# baseline (speedup 1.0000x reference)
import jax
import jax.numpy as jnp
from jax import lax
from jax.experimental import pallas as pl
from jax.experimental.pallas import tpu as pltpu
from jax.experimental.pallas import tpu_sc as plsc

_NC = 2
_NS = 16
_L = 16
_NW = _NC * _NS

_V = 1000
_C = 1000
_N = 51200
_PB = _C // 8
_PT = _N // 128
_RPW = _N // _NW
_EG = 80
_NEG = _RPW // _EG


def _lse_body(tab_ref, lse_ref):
    x = tab_ref[...]
    m = jnp.max(x, axis=1)
    s = jnp.sum(jnp.exp(x - m[:, None]), axis=1)
    lse_ref[...] = m + jnp.log(s)


def _loss_body(part_ref, loss_ref):
    loss_ref[...] = jnp.sum(part_ref[...], axis=(0, 1), keepdims=True) * (1.0 / _N)


def _tgather_body(arr4_hbm, inp_hbm, tflat_hbm, tgt_hbm, lse_hbm,
                  out_hbm, part_hbm,
                  inp_v, tt_v, st_v, tgt_v, pr_v, tl_v, lse_v, acc_v,
                  ssem0, ssem1, gsem):
    wid = lax.axis_index("s") * _NC + lax.axis_index("c")

    pltpu.sync_copy(inp_hbm, inp_v)
    ssems = (ssem0, ssem1)

    base = wid * _RPW
    pltpu.sync_copy(tgt_hbm.at[pl.ds(base, _RPW)], tgt_v)
    pltpu.sync_copy(lse_hbm, lse_v)

    def mk_pairs(i, carry):
        sl = pl.ds(i * _L, _L)
        pr_v[sl] = inp_v[pl.ds(base + i * _L, _L)] * _V + tgt_v[sl]
        return carry
    lax.fori_loop(0, _RPW // _L, mk_pairs, 0)

    def fire(g, carry):
        pltpu.async_copy(
            tflat_hbm.at[pr_v.at[pl.ds(g * _EG, _EG)]],
            tl_v.at[pl.ds(g * _EG, _EG)], gsem)
        return carry
    lax.fori_loop(0, _NEG, fire, 0)

    def drain(g, carry):
        pltpu.make_async_copy(
            tflat_hbm.at[pr_v.at[pl.ds(0, _EG)]],
            tl_v.at[pl.ds(g * _EG, _EG)], gsem).wait()
        return carry
    lax.fori_loop(0, _NEG, drain, 0)

    for z in range(8):
        acc_v[pl.ds(z * _L, _L)] = jnp.zeros((_L,), jnp.float32)

    def accum(i, carry):
        sl = pl.ds(i * _L, _L)
        ivec = inp_v[pl.ds(base + i * _L, _L)]
        ls = plsc.load_gather(lse_v, [ivec])
        acc_v[pl.ds(0, _L)] = acc_v[pl.ds(0, _L)] + (ls - tl_v[sl])
        return carry
    lax.fori_loop(0, _RPW // _L, accum, 0)

    pltpu.sync_copy(acc_v, part_hbm.at[wid])

    def do_band(k, carry):
        b = wid + _NW * k

        @pl.when(b < _PB)
        def _():
            pltpu.sync_copy(arr4_hbm.at[b], tt_v)

            def dma_start(pt, buf):
                pltpu.async_copy(
                    st_v.at[buf],
                    out_hbm.at[pl.ds(b * 8, 8), pl.ds(pt * 128, 128)],
                    ssems[buf])

            def dma_wait(buf):
                pltpu.make_async_copy(
                    st_v.at[buf],
                    out_hbm.at[pl.ds(0, 8), pl.ds(0, 128)],
                    ssems[buf]).wait()

            def do_pt2(q, c2):
                for buf in range(2):
                    pt = q * 2 + buf

                    @pl.when(q > 0)
                    def _():
                        dma_wait(buf)

                    @plsc.parallel_loop(0, 8, 1, unroll=8)
                    def _groups(i):
                        ivec = inp_v[pl.ds(pt * 128 + i * _L, _L)]
                        zero = jnp.zeros((_L,), jnp.int32)
                        base = lax.bitwise_or(
                            lax.shift_left(
                                lax.shift_right_logical(ivec, 7), 10),
                            lax.bitwise_and(ivec, 127))
                        for cl in range(8):
                            vals = plsc.load_gather(
                                tt_v, [zero, zero, base + cl * 128])
                            st_v[buf, cl, pl.ds(i * _L, _L)] = vals

                    dma_start(pt, buf)
                return c2

            lax.fori_loop(0, _PT // 2, do_pt2, 0)
            dma_wait(0)
            dma_wait(1)

        return carry

    lax.fori_loop(0, 4, do_band, 0)


def _lpart_body(tflat_hbm, inp_hbm, tgt_hbm, lse_hbm,
                part_hbm,
                idx_v, tgt_v, pr_v, tl_v, lse_v, acc_v, gsem):
    wid = lax.axis_index("s") * _NC + lax.axis_index("c")
    base = wid * _RPW

    pltpu.sync_copy(inp_hbm.at[pl.ds(base, _RPW)], idx_v)
    pltpu.sync_copy(tgt_hbm.at[pl.ds(base, _RPW)], tgt_v)
    pltpu.sync_copy(lse_hbm, lse_v)

    def mk_pairs(i, carry):
        sl = pl.ds(i * _L, _L)
        pr_v[sl] = idx_v[sl] * _V + tgt_v[sl]
        return carry
    lax.fori_loop(0, _RPW // _L, mk_pairs, 0)

    def fire(g, carry):
        pltpu.async_copy(
            tflat_hbm.at[pr_v.at[pl.ds(g * _EG, _EG)]],
            tl_v.at[pl.ds(g * _EG, _EG)], gsem)
        return carry
    lax.fori_loop(0, _NEG, fire, 0)

    def drain(g, carry):
        pltpu.make_async_copy(
            tflat_hbm.at[pr_v.at[pl.ds(0, _EG)]],
            tl_v.at[pl.ds(g * _EG, _EG)], gsem).wait()
        return carry
    lax.fori_loop(0, _NEG, drain, 0)

    acc_v[...] = jnp.zeros((_L,), jnp.float32)

    def accum(i, carry):
        sl = pl.ds(i * _L, _L)
        ivec = idx_v[sl]
        ls = plsc.load_gather(lse_v, [ivec])
        acc_v[...] = acc_v[...] + (ls - tl_v[sl])
        return carry
    lax.fori_loop(0, _RPW // _L, accum, 0)

    pltpu.sync_copy(acc_v, part_hbm.at[wid])


def kernel(input_sequences, target_sequences, token_embedding_table):
    inp = input_sequences.reshape(-1)
    tgt = target_sequences.reshape(-1)

    lse = pl.pallas_call(
        _lse_body,
        out_shape=jax.ShapeDtypeStruct((_V,), jnp.float32),
    )(token_embedding_table)
    lse_pad = jnp.pad(lse, (0, 1024 - _V))

    table_t = jnp.pad(token_embedding_table.T, ((0, 0), (0, 24)))
    arr4 = table_t.reshape(_PB, 8, 8, 128).transpose(0, 2, 1, 3)
    tflat = token_embedding_table.reshape(-1)

    mesh = plsc.VectorSubcoreMesh(
        core_axis_name="c", subcore_axis_name="s",
        num_cores=_NC, num_subcores=_NS)

    out_t, partials = pl.kernel(
        _tgather_body,
        out_type=[
            jax.ShapeDtypeStruct((_C, _N), jnp.float32),
            jax.ShapeDtypeStruct((_NW, 128), jnp.float32),
        ],
        mesh=mesh,
        compiler_params=pltpu.CompilerParams(
            needs_layout_passes=False, use_tc_tiling_on_sc=True),
        scratch_types=[
            pltpu.VMEM((_N,), jnp.int32),
            pltpu.VMEM((8, 8, 128), jnp.float32),
            pltpu.VMEM((2, 8, 128), jnp.float32),
            pltpu.VMEM((_RPW,), jnp.int32),
            pltpu.VMEM((_RPW,), jnp.int32),
            pltpu.VMEM((_RPW,), jnp.float32),
            pltpu.VMEM((1024,), jnp.float32),
            pltpu.VMEM((128,), jnp.float32),
            pltpu.SemaphoreType.DMA,
            pltpu.SemaphoreType.DMA,
            pltpu.SemaphoreType.DMA,
        ],
    )(arr4, inp, tflat, tgt, lse_pad)

    logits_flat = out_t.T

    loss2d = pl.pallas_call(
        _loss_body,
        out_shape=jax.ShapeDtypeStruct((1, 1), jnp.float32),
    )(partials)
    return logits_flat, loss2d[0, 0]


# speedup vs baseline: 1.0150x; 1.0150x over previous
import jax
import jax.numpy as jnp
from jax import lax
from jax.experimental import pallas as pl
from jax.experimental.pallas import tpu as pltpu
from jax.experimental.pallas import tpu_sc as plsc

_NC = 2
_NS = 16
_L = 16
_NW = _NC * _NS

_V = 1000
_C = 1000
_N = 51200
_PB = _C // 8
_PT = _N // 128
_RPW = _N // _NW
_EG = 80
_NEG = _RPW // _EG


def _lse_body(tab_ref, lse_ref):
    x = tab_ref[...]
    m = jnp.max(x, axis=1)
    s = jnp.sum(jnp.exp(x - m[:, None]), axis=1)
    lse_ref[...] = m + jnp.log(s)


def _loss_body(part_ref, loss_ref):
    loss_ref[...] = jnp.sum(part_ref[...], axis=(0, 1), keepdims=True) * (1.0 / _N)


def _tgather_body(arr4_hbm, inp_hbm, out_hbm,
                  inp_v, tt_v, st_v, ssem0, ssem1):
    wid = lax.axis_index("s") * _NC + lax.axis_index("c")

    pltpu.sync_copy(inp_hbm, inp_v)
    ssems = (ssem0, ssem1)

    def do_band(k, carry):
        b = wid + _NW * k

        @pl.when(b < _PB)
        def _():
            pltpu.sync_copy(arr4_hbm.at[b], tt_v)

            def dma_start(pt, buf):
                pltpu.async_copy(
                    st_v.at[buf],
                    out_hbm.at[pl.ds(b * 8, 8), pl.ds(pt * 128, 128)],
                    ssems[buf])

            def dma_wait(buf):
                pltpu.make_async_copy(
                    st_v.at[buf],
                    out_hbm.at[pl.ds(0, 8), pl.ds(0, 128)],
                    ssems[buf]).wait()

            def do_pt2(q, c2):
                for buf in range(2):
                    pt = q * 2 + buf

                    @pl.when(q > 0)
                    def _():
                        dma_wait(buf)

                    @plsc.parallel_loop(0, 8, 1, unroll=8)
                    def _groups(i):
                        ivec = inp_v[pl.ds(pt * 128 + i * _L, _L)]
                        zero = jnp.zeros((_L,), jnp.int32)
                        base = lax.bitwise_or(
                            lax.shift_left(
                                lax.shift_right_logical(ivec, 7), 10),
                            lax.bitwise_and(ivec, 127))
                        for cl in range(8):
                            vals = plsc.load_gather(
                                tt_v, [zero, zero, base + cl * 128])
                            st_v[buf, cl, pl.ds(i * _L, _L)] = vals

                    dma_start(pt, buf)
                return c2

            lax.fori_loop(0, _PT // 2, do_pt2, 0)
            dma_wait(0)
            dma_wait(1)

        return carry

    lax.fori_loop(0, 4, do_band, 0)


def _lpart_body(tflat_hbm, inp_hbm, tgt_hbm, lse_hbm,
                part_hbm,
                idx_v, tgt_v, pr_v, tl_v, lse_v, acc_v, gsem):
    wid = lax.axis_index("s") * _NC + lax.axis_index("c")
    base = wid * _RPW

    pltpu.sync_copy(inp_hbm.at[pl.ds(base, _RPW)], idx_v)
    pltpu.sync_copy(tgt_hbm.at[pl.ds(base, _RPW)], tgt_v)
    pltpu.sync_copy(lse_hbm, lse_v)

    def mk_pairs(i, carry):
        sl = pl.ds(i * _L, _L)
        pr_v[sl] = idx_v[sl] * _V + tgt_v[sl]
        return carry
    lax.fori_loop(0, _RPW // _L, mk_pairs, 0)

    def fire(g, carry):
        pltpu.async_copy(
            tflat_hbm.at[pr_v.at[pl.ds(g * _EG, _EG)]],
            tl_v.at[pl.ds(g * _EG, _EG)], gsem)
        return carry
    lax.fori_loop(0, _NEG, fire, 0)

    def drain(g, carry):
        pltpu.make_async_copy(
            tflat_hbm.at[pr_v.at[pl.ds(0, _EG)]],
            tl_v.at[pl.ds(g * _EG, _EG)], gsem).wait()
        return carry
    lax.fori_loop(0, _NEG, drain, 0)

    acc_v[...] = jnp.zeros((_L,), jnp.float32)

    def accum(i, carry):
        sl = pl.ds(i * _L, _L)
        ivec = idx_v[sl]
        ls = plsc.load_gather(lse_v, [ivec])
        acc_v[...] = acc_v[...] + (ls - tl_v[sl])
        return carry
    lax.fori_loop(0, _RPW // _L, accum, 0)

    pltpu.sync_copy(acc_v, part_hbm.at[wid])


def kernel(input_sequences, target_sequences, token_embedding_table):
    inp = input_sequences.reshape(-1)
    tgt = target_sequences.reshape(-1)

    lse = pl.pallas_call(
        _lse_body,
        out_shape=jax.ShapeDtypeStruct((_V,), jnp.float32),
    )(token_embedding_table)
    lse_pad = jnp.pad(lse, (0, 1024 - _V))

    table_t = jnp.pad(token_embedding_table.T, ((0, 0), (0, 24)))
    arr4 = table_t.reshape(_PB, 8, 8, 128).transpose(0, 2, 1, 3)
    tflat = token_embedding_table.reshape(-1)

    mesh = plsc.VectorSubcoreMesh(
        core_axis_name="c", subcore_axis_name="s",
        num_cores=_NC, num_subcores=_NS)

    out_t = pl.kernel(
        _tgather_body,
        out_type=jax.ShapeDtypeStruct((_C, _N), jnp.float32),
        mesh=mesh,
        compiler_params=pltpu.CompilerParams(
            needs_layout_passes=False, use_tc_tiling_on_sc=True),
        scratch_types=[
            pltpu.VMEM((_N,), jnp.int32),
            pltpu.VMEM((8, 8, 128), jnp.float32),
            pltpu.VMEM((2, 8, 128), jnp.float32),
            pltpu.SemaphoreType.DMA,
            pltpu.SemaphoreType.DMA,
        ],
    )(arr4, inp)

    logits_flat = out_t.T

    partials = pl.kernel(
        _lpart_body,
        out_type=jax.ShapeDtypeStruct((_NW, _L), jnp.float32),
        mesh=mesh,
        compiler_params=pltpu.CompilerParams(
            needs_layout_passes=False, use_tc_tiling_on_sc=False),
        scratch_types=[
            pltpu.VMEM((_RPW,), jnp.int32),
            pltpu.VMEM((_RPW,), jnp.int32),
            pltpu.VMEM((_RPW,), jnp.int32),
            pltpu.VMEM((_RPW,), jnp.float32),
            pltpu.VMEM((1024,), jnp.float32),
            pltpu.VMEM((_L,), jnp.float32),
            pltpu.SemaphoreType.DMA,
        ],
    )(tflat, inp, tgt, lse_pad)

    loss2d = pl.pallas_call(
        _loss_body,
        out_shape=jax.ShapeDtypeStruct((1, 1), jnp.float32),
    )(partials)
    return logits_flat, loss2d[0, 0]


# speedup vs baseline: 1.3651x; 1.3449x over previous
import jax
import jax.numpy as jnp
from jax import lax
from jax.experimental import pallas as pl
from jax.experimental.pallas import tpu as pltpu
from jax.experimental.pallas import tpu_sc as plsc

_NC = 2
_NS = 16
_L = 16
_NW = _NC * _NS

_V = 1000
_C = 1000
_N = 51200
_PB = _C // 8
_PT = _N // 128
_RPW = _N // _NW
_EG = 80
_NEG = _RPW // _EG


def _lse_body(tab_ref, lse_ref):
    x = tab_ref[...]
    m = jnp.max(x, axis=1)
    s = jnp.sum(jnp.exp(x - m[:, None]), axis=1)
    lse_ref[...] = m + jnp.log(s)


def _loss_body(part_ref, loss_ref):
    loss_ref[...] = jnp.sum(part_ref[...], axis=(0, 1), keepdims=True) * (1.0 / _N)


def _tgather_body(arr4_hbm, inp_hbm, out_hbm,
                  inp_v, tt_v, st_v, ssem0, ssem1):
    wid = lax.axis_index("s") * _NC + lax.axis_index("c")

    pltpu.sync_copy(inp_hbm, inp_v)
    ssems = (ssem0, ssem1)

    def do_band(k, carry):
        b = wid + _NW * k

        @pl.when(b < _PB)
        def _():
            pltpu.sync_copy(arr4_hbm.at[b], tt_v)

            def dma_start(ck, buf):
                pltpu.async_copy(
                    st_v.at[buf],
                    out_hbm.at[b, pl.ds(ck * 4, 4)],
                    ssems[buf])

            def dma_wait(buf):
                pltpu.make_async_copy(
                    st_v.at[buf],
                    out_hbm.at[0, pl.ds(0, 4)],
                    ssems[buf]).wait()

            def do_ck2(q, c2):
                for buf in range(2):
                    ck = q * 2 + buf

                    @pl.when(q > 0)
                    def _():
                        dma_wait(buf)

                    @plsc.parallel_loop(0, 32, 1, unroll=8)
                    def _groups(i):
                        ivec = inp_v[pl.ds(ck * 512 + i * _L, _L)]
                        zero = jnp.zeros((_L,), jnp.int32)
                        base = lax.bitwise_or(
                            lax.shift_left(
                                lax.shift_right_logical(ivec, 7), 10),
                            lax.bitwise_and(ivec, 127))
                        tix = i // 8
                        lo = (i % 8) * _L
                        for cl in range(8):
                            vals = plsc.load_gather(
                                tt_v, [zero, zero, base + cl * 128])
                            st_v[buf, tix, cl, pl.ds(lo, _L)] = vals

                    dma_start(ck, buf)
                return c2

            lax.fori_loop(0, _PT // 8, do_ck2, 0)
            dma_wait(0)
            dma_wait(1)

        return carry

    lax.fori_loop(0, 4, do_band, 0)


def _lpart_body(tflat_hbm, inp_hbm, tgt_hbm, lse_hbm,
                part_hbm,
                idx_v, tgt_v, pr_v, tl_v, lse_v, acc_v, gsem):
    wid = lax.axis_index("s") * _NC + lax.axis_index("c")
    base = wid * _RPW

    pltpu.sync_copy(inp_hbm.at[pl.ds(base, _RPW)], idx_v)
    pltpu.sync_copy(tgt_hbm.at[pl.ds(base, _RPW)], tgt_v)
    pltpu.sync_copy(lse_hbm, lse_v)

    def mk_pairs(i, carry):
        sl = pl.ds(i * _L, _L)
        pr_v[sl] = idx_v[sl] * _V + tgt_v[sl]
        return carry
    lax.fori_loop(0, _RPW // _L, mk_pairs, 0)

    def fire(g, carry):
        pltpu.async_copy(
            tflat_hbm.at[pr_v.at[pl.ds(g * _EG, _EG)]],
            tl_v.at[pl.ds(g * _EG, _EG)], gsem)
        return carry
    lax.fori_loop(0, _NEG, fire, 0)

    def drain(g, carry):
        pltpu.make_async_copy(
            tflat_hbm.at[pr_v.at[pl.ds(0, _EG)]],
            tl_v.at[pl.ds(g * _EG, _EG)], gsem).wait()
        return carry
    lax.fori_loop(0, _NEG, drain, 0)

    acc_v[...] = jnp.zeros((_L,), jnp.float32)

    def accum(i, carry):
        sl = pl.ds(i * _L, _L)
        ivec = idx_v[sl]
        ls = plsc.load_gather(lse_v, [ivec])
        acc_v[...] = acc_v[...] + (ls - tl_v[sl])
        return carry
    lax.fori_loop(0, _RPW // _L, accum, 0)

    pltpu.sync_copy(acc_v, part_hbm.at[wid])


def kernel(input_sequences, target_sequences, token_embedding_table):
    inp = input_sequences.reshape(-1)
    tgt = target_sequences.reshape(-1)

    lse = pl.pallas_call(
        _lse_body,
        out_shape=jax.ShapeDtypeStruct((_V,), jnp.float32),
    )(token_embedding_table)
    lse_pad = jnp.pad(lse, (0, 1024 - _V))

    table_t = jnp.pad(token_embedding_table.T, ((0, 0), (0, 24)))
    arr4 = table_t.reshape(_PB, 8, 8, 128).transpose(0, 2, 1, 3)
    tflat = token_embedding_table.reshape(-1)

    mesh = plsc.VectorSubcoreMesh(
        core_axis_name="c", subcore_axis_name="s",
        num_cores=_NC, num_subcores=_NS)

    out4d = pl.kernel(
        _tgather_body,
        out_type=jax.ShapeDtypeStruct((_PB, _PT, 8, 128), jnp.float32),
        mesh=mesh,
        compiler_params=pltpu.CompilerParams(
            needs_layout_passes=False, use_tc_tiling_on_sc=True),
        scratch_types=[
            pltpu.VMEM((_N,), jnp.int32),
            pltpu.VMEM((8, 8, 128), jnp.float32),
            pltpu.VMEM((2, 4, 8, 128), jnp.float32),
            pltpu.SemaphoreType.DMA,
            pltpu.SemaphoreType.DMA,
        ],
    )(arr4, inp)

    logits_flat = out4d.transpose(1, 3, 0, 2).reshape(_N, _C)

    partials = pl.kernel(
        _lpart_body,
        out_type=jax.ShapeDtypeStruct((_NW, _L), jnp.float32),
        mesh=mesh,
        compiler_params=pltpu.CompilerParams(
            needs_layout_passes=False, use_tc_tiling_on_sc=False),
        scratch_types=[
            pltpu.VMEM((_RPW,), jnp.int32),
            pltpu.VMEM((_RPW,), jnp.int32),
            pltpu.VMEM((_RPW,), jnp.int32),
            pltpu.VMEM((_RPW,), jnp.float32),
            pltpu.VMEM((1024,), jnp.float32),
            pltpu.VMEM((_L,), jnp.float32),
            pltpu.SemaphoreType.DMA,
        ],
    )(tflat, inp, tgt, lse_pad)

    loss2d = pl.pallas_call(
        _loss_body,
        out_shape=jax.ShapeDtypeStruct((1, 1), jnp.float32),
    )(partials)
    return logits_flat, loss2d[0, 0]


# speedup vs baseline: 1.4214x; 1.0413x over previous
import jax
import jax.numpy as jnp
from jax import lax
from jax.experimental import pallas as pl
from jax.experimental.pallas import tpu as pltpu
from jax.experimental.pallas import tpu_sc as plsc

_NC = 2
_NS = 16
_L = 16
_NW = _NC * _NS

_V = 1000
_C = 1000
_N = 51200
_PB = _C // 8
_PT = _N // 128
_RPW = _N // _NW
_EG = 80
_NEG = _RPW // _EG


def _lse_body(tab_ref, lse_ref):
    x = tab_ref[...]
    m = jnp.max(x, axis=1)
    s = jnp.sum(jnp.exp(x - m[:, None]), axis=1)
    lse_ref[...] = m + jnp.log(s)


def _loss_body(part_ref, loss_ref):
    loss_ref[...] = jnp.sum(part_ref[...], axis=(0, 1), keepdims=True) * (1.0 / _N)


def _tgather_body(arr4_hbm, inp_hbm, out_hbm,
                  inp_v, tt_v, st_v, ssem0, ssem1):
    wid = lax.axis_index("s") * _NC + lax.axis_index("c")

    pltpu.sync_copy(inp_hbm, inp_v)
    ssems = (ssem0, ssem1)

    def do_band(k, carry):
        b = wid + _NW * k

        @pl.when(b < _PB)
        def _():
            pltpu.sync_copy(arr4_hbm.at[b], tt_v)

            def dma_start(ck, buf):
                pltpu.async_copy(
                    st_v.at[buf],
                    out_hbm.at[b, pl.ds(ck * 8, 8)],
                    ssems[buf])

            def dma_wait(buf):
                pltpu.make_async_copy(
                    st_v.at[buf],
                    out_hbm.at[0, pl.ds(0, 8)],
                    ssems[buf]).wait()

            def do_ck2(q, c2):
                for buf in range(2):
                    ck = q * 2 + buf

                    @pl.when(q > 0)
                    def _():
                        dma_wait(buf)

                    @plsc.parallel_loop(0, 64, 1, unroll=8)
                    def _groups(i):
                        ivec = inp_v[pl.ds(ck * 1024 + i * _L, _L)]
                        zero = jnp.zeros((_L,), jnp.int32)
                        base = lax.bitwise_or(
                            lax.shift_left(
                                lax.shift_right_logical(ivec, 7), 10),
                            lax.bitwise_and(ivec, 127))
                        tix = i // 8
                        lo = (i % 8) * _L
                        for cl in range(8):
                            vals = plsc.load_gather(
                                tt_v, [zero, zero, base + cl * 128])
                            st_v[buf, tix, cl, pl.ds(lo, _L)] = vals

                    dma_start(ck, buf)
                return c2

            lax.fori_loop(0, _PT // 16, do_ck2, 0)
            dma_wait(0)
            dma_wait(1)

        return carry

    lax.fori_loop(0, 4, do_band, 0)


def _lpart_body(tflat_hbm, inp_hbm, tgt_hbm, lse_hbm,
                part_hbm,
                idx_v, tgt_v, pr_v, tl_v, lse_v, acc_v, gsem):
    wid = lax.axis_index("s") * _NC + lax.axis_index("c")
    base = wid * _RPW

    pltpu.sync_copy(inp_hbm.at[pl.ds(base, _RPW)], idx_v)
    pltpu.sync_copy(tgt_hbm.at[pl.ds(base, _RPW)], tgt_v)
    pltpu.sync_copy(lse_hbm, lse_v)

    def mk_pairs(i, carry):
        sl = pl.ds(i * _L, _L)
        pr_v[sl] = idx_v[sl] * _V + tgt_v[sl]
        return carry
    lax.fori_loop(0, _RPW // _L, mk_pairs, 0)

    def fire(g, carry):
        pltpu.async_copy(
            tflat_hbm.at[pr_v.at[pl.ds(g * _EG, _EG)]],
            tl_v.at[pl.ds(g * _EG, _EG)], gsem)
        return carry
    lax.fori_loop(0, _NEG, fire, 0)

    def drain(g, carry):
        pltpu.make_async_copy(
            tflat_hbm.at[pr_v.at[pl.ds(0, _EG)]],
            tl_v.at[pl.ds(g * _EG, _EG)], gsem).wait()
        return carry
    lax.fori_loop(0, _NEG, drain, 0)

    acc_v[...] = jnp.zeros((_L,), jnp.float32)

    def accum(i, carry):
        sl = pl.ds(i * _L, _L)
        ivec = idx_v[sl]
        ls = plsc.load_gather(lse_v, [ivec])
        acc_v[...] = acc_v[...] + (ls - tl_v[sl])
        return carry
    lax.fori_loop(0, _RPW // _L, accum, 0)

    pltpu.sync_copy(acc_v, part_hbm.at[wid])


def kernel(input_sequences, target_sequences, token_embedding_table):
    inp = input_sequences.reshape(-1)
    tgt = target_sequences.reshape(-1)

    lse = pl.pallas_call(
        _lse_body,
        out_shape=jax.ShapeDtypeStruct((_V,), jnp.float32),
    )(token_embedding_table)
    lse_pad = jnp.pad(lse, (0, 1024 - _V))

    table_t = jnp.pad(token_embedding_table.T, ((0, 0), (0, 24)))
    arr4 = table_t.reshape(_PB, 8, 8, 128).transpose(0, 2, 1, 3)
    tflat = token_embedding_table.reshape(-1)

    mesh = plsc.VectorSubcoreMesh(
        core_axis_name="c", subcore_axis_name="s",
        num_cores=_NC, num_subcores=_NS)

    out4d = pl.kernel(
        _tgather_body,
        out_type=jax.ShapeDtypeStruct((_PB, _PT, 8, 128), jnp.float32),
        mesh=mesh,
        compiler_params=pltpu.CompilerParams(
            needs_layout_passes=False, use_tc_tiling_on_sc=True),
        scratch_types=[
            pltpu.VMEM((_N,), jnp.int32),
            pltpu.VMEM((8, 8, 128), jnp.float32),
            pltpu.VMEM((2, 8, 8, 128), jnp.float32),
            pltpu.SemaphoreType.DMA,
            pltpu.SemaphoreType.DMA,
        ],
    )(arr4, inp)

    logits_flat = out4d.transpose(1, 3, 0, 2).reshape(_N, _C)

    partials = pl.kernel(
        _lpart_body,
        out_type=jax.ShapeDtypeStruct((_NW, _L), jnp.float32),
        mesh=mesh,
        compiler_params=pltpu.CompilerParams(
            needs_layout_passes=False, use_tc_tiling_on_sc=False),
        scratch_types=[
            pltpu.VMEM((_RPW,), jnp.int32),
            pltpu.VMEM((_RPW,), jnp.int32),
            pltpu.VMEM((_RPW,), jnp.int32),
            pltpu.VMEM((_RPW,), jnp.float32),
            pltpu.VMEM((1024,), jnp.float32),
            pltpu.VMEM((_L,), jnp.float32),
            pltpu.SemaphoreType.DMA,
        ],
    )(tflat, inp, tgt, lse_pad)

    loss2d = pl.pallas_call(
        _loss_body,
        out_shape=jax.ShapeDtypeStruct((1, 1), jnp.float32),
    )(partials)
    return logits_flat, loss2d[0, 0]
